# Initial kernel scaffold; baseline (speedup 1.0000x reference)
import jax
import jax.numpy as jnp
from jax.experimental import pallas as pl


def kernel(x, edge_index, W_q, keys_mem, values_mem, beta, ln_gamma, ln_beta):
    raise NotImplementedError("write your pallas kernel here")


# speedup vs baseline: 4.4689x; 4.4689x over previous
import functools

import jax
import jax.numpy as jnp
from jax import lax
from jax.experimental import pallas as pl
from jax.experimental.pallas import tpu as pltpu
from jax.experimental.pallas import tpu_sc as plsc

NUM_ITERATIONS = 2
ALPHA = 0.5
LAMBDA_GRAPH = 0.1
LN_EPS = 1e-5

NC = 2
NS = 16
NW = NC * NS


def _sc_scatter(row3, col3, y, zeros_nd):
    NCHUNK, CH = row3.shape[1], row3.shape[2]
    N, D = y.shape
    RB = 1000
    NRB = N // RB

    @functools.partial(
        pl.kernel,
        out_type=jax.ShapeDtypeStruct((NC, N, D), jnp.float32),
        mesh=plsc.VectorSubcoreMesh(core_axis_name="c", subcore_axis_name="s"),
        scratch_types=[
            pltpu.VMEM((NCHUNK, CH), jnp.int32),
            pltpu.VMEM((NCHUNK, CH), jnp.int32),
            pltpu.VMEM((CH, D), jnp.float32),
            pltpu.VMEM_SHARED((N, D), jnp.float32),
            pltpu.SemaphoreType.DMA,
        ],
    )
    def k(row_hbm, col_hbm, y_hbm, z_hbm, out_hbm, idxr, idxc, buf, acc, gsem):
        c = lax.axis_index("c")
        s = lax.axis_index("s")
        wid = c * NS + s
        pltpu.sync_copy(row_hbm.at[wid], idxr)
        pltpu.sync_copy(col_hbm.at[wid], idxc)

        @pl.when(s < NRB)
        def _():
            pltpu.sync_copy(z_hbm.at[pl.ds(s * RB, RB)],
                            acc.at[pl.ds(s * RB, RB)])

        plsc.subcore_barrier()

        def body(j, carry):
            pltpu.async_copy(y_hbm.at[idxc.at[j]], buf, gsem).wait()
            pltpu.sync_copy(buf, acc.at[idxr.at[j]], add=True)
            return carry

        lax.fori_loop(0, NCHUNK, body, 0)
        plsc.subcore_barrier()

        @pl.when(s < NRB)
        def _():
            pltpu.sync_copy(acc.at[pl.ds(s * RB, RB)],
                            out_hbm.at[c, pl.ds(s * RB, RB)])

    return k(row3, col3, y, zeros_nd)


def _retrieve_block(xb, wq, keys, values, beta):
    q = jnp.dot(xb, wq, preferred_element_type=jnp.float32)
    logits = beta * lax.dot_general(
        q, keys, (((1,), (1,)), ((), ())), preferred_element_type=jnp.float32)
    m = jnp.max(logits, axis=-1, keepdims=True)
    e = jnp.exp(logits - m)
    attn = e / jnp.sum(e, axis=-1, keepdims=True)
    return jnp.dot(attn, values, preferred_element_type=jnp.float32)


def _tc_prep(x, degp, W_q, keys, values, beta2, block_n):
    N, D = x.shape
    K = keys.shape[0]

    def body(x_ref, deg_ref, wq_ref, k_ref, v_ref, b_ref,
             dis_ref, y_ref, r_ref):
        xb = x_ref[...]
        deg = deg_ref[0, :, 0:1] + deg_ref[1, :, 0:1]
        dis = jnp.where(deg > 0, lax.rsqrt(deg), 0.0)
        dis_ref[...] = dis
        y_ref[...] = xb * dis
        r_ref[...] = _retrieve_block(xb, wq_ref[...], k_ref[...], v_ref[...],
                                     b_ref[0, 0])

    return pl.pallas_call(
        body,
        grid=(N // block_n,),
        in_specs=[
            pl.BlockSpec((block_n, D), lambda i: (i, 0)),
            pl.BlockSpec((NC, block_n, D), lambda i: (0, i, 0)),
            pl.BlockSpec((D, D), lambda i: (0, 0)),
            pl.BlockSpec((K, D), lambda i: (0, 0)),
            pl.BlockSpec((K, D), lambda i: (0, 0)),
            pl.BlockSpec(memory_space=pltpu.SMEM),
        ],
        out_specs=[
            pl.BlockSpec((block_n, 1), lambda i: (i, 0)),
            pl.BlockSpec((block_n, D), lambda i: (i, 0)),
            pl.BlockSpec((block_n, D), lambda i: (i, 0)),
        ],
        out_shape=[
            jax.ShapeDtypeStruct((N, 1), jnp.float32),
            jax.ShapeDtypeStruct((N, D), jnp.float32),
            jax.ShapeDtypeStruct((N, D), jnp.float32),
        ],
    )(x, degp, W_q, keys, values, beta2)


def _update_block(xb, rb, aggp, dis):
    agg = dis * (aggp[0] + aggp[1])
    lap = xb - agg
    return (1.0 - ALPHA) * xb + ALPHA * (rb - 2.0 * LAMBDA_GRAPH * lap)


def _tc_update_retrieve(x, r, aggp, dis, W_q, keys, values, beta2, block_n):
    N, D = x.shape
    K = keys.shape[0]

    def body(x_ref, r_ref, agg_ref, dis_ref, wq_ref, k_ref, v_ref, b_ref,
             xn_ref, y_ref, rn_ref):
        dis = dis_ref[...]
        xn = _update_block(x_ref[...], r_ref[...], agg_ref[...], dis)
        xn_ref[...] = xn
        y_ref[...] = xn * dis
        rn_ref[...] = _retrieve_block(xn, wq_ref[...], k_ref[...], v_ref[...],
                                      b_ref[0, 0])

    return pl.pallas_call(
        body,
        grid=(N // block_n,),
        in_specs=[
            pl.BlockSpec((block_n, D), lambda i: (i, 0)),
            pl.BlockSpec((block_n, D), lambda i: (i, 0)),
            pl.BlockSpec((NC, block_n, D), lambda i: (0, i, 0)),
            pl.BlockSpec((block_n, 1), lambda i: (i, 0)),
            pl.BlockSpec((D, D), lambda i: (0, 0)),
            pl.BlockSpec((K, D), lambda i: (0, 0)),
            pl.BlockSpec((K, D), lambda i: (0, 0)),
            pl.BlockSpec(memory_space=pltpu.SMEM),
        ],
        out_specs=[
            pl.BlockSpec((block_n, D), lambda i: (i, 0)),
            pl.BlockSpec((block_n, D), lambda i: (i, 0)),
            pl.BlockSpec((block_n, D), lambda i: (i, 0)),
        ],
        out_shape=[
            jax.ShapeDtypeStruct((N, D), jnp.float32),
            jax.ShapeDtypeStruct((N, D), jnp.float32),
            jax.ShapeDtypeStruct((N, D), jnp.float32),
        ],
    )(x, r, aggp, dis, W_q, keys, values, beta2)


def _tc_final(x, r, aggp, dis, ln_gamma, ln_beta, block_n):
    N, D = x.shape

    def body(x_ref, r_ref, agg_ref, dis_ref, g_ref, b_ref, out_ref):
        xn = _update_block(x_ref[...], r_ref[...], agg_ref[...], dis_ref[...])
        mean = jnp.mean(xn, axis=-1, keepdims=True)
        d = xn - mean
        var = jnp.mean(d * d, axis=-1, keepdims=True)
        out_ref[...] = g_ref[...] * d * lax.rsqrt(var + LN_EPS) + b_ref[...]

    return pl.pallas_call(
        body,
        grid=(N // block_n,),
        in_specs=[
            pl.BlockSpec((block_n, D), lambda i: (i, 0)),
            pl.BlockSpec((block_n, D), lambda i: (i, 0)),
            pl.BlockSpec((NC, block_n, D), lambda i: (0, i, 0)),
            pl.BlockSpec((block_n, 1), lambda i: (i, 0)),
            pl.BlockSpec((D,), lambda i: (0,)),
            pl.BlockSpec((D,), lambda i: (0,)),
        ],
        out_specs=pl.BlockSpec((block_n, D), lambda i: (i, 0)),
        out_shape=jax.ShapeDtypeStruct((N, D), jnp.float32),
    )(x, r, aggp, dis, ln_gamma, ln_beta)


def kernel(x, edge_index, W_q, keys_mem, values_mem, beta, ln_gamma, ln_beta):
    N, D = x.shape
    E = edge_index.shape[1]
    EPW = E // NW
    CH = 80
    NCHUNK = EPW // CH

    ei = edge_index.astype(jnp.int32)
    row3 = ei[0].reshape(NW, NCHUNK, CH)
    col3 = ei[1].reshape(NW, NCHUNK, CH)
    zeros_nd = jnp.zeros((N, D), jnp.float32)
    ones_nd = jnp.ones((N, D), jnp.float32)
    beta2 = jnp.reshape(beta.astype(jnp.float32), (1, 1))
    block_n = 2000

    degp = _sc_scatter(col3, col3, ones_nd, zeros_nd)
    dis, y, r = _tc_prep(x, degp, W_q, keys_mem, values_mem, beta2, block_n)
    for it in range(NUM_ITERATIONS):
        aggp = _sc_scatter(row3, col3, y, zeros_nd)
        if it + 1 < NUM_ITERATIONS:
            x, y, r = _tc_update_retrieve(
                x, r, aggp, dis, W_q, keys_mem, values_mem, beta2, block_n)
        else:
            out = _tc_final(x, r, aggp, dis, ln_gamma, ln_beta, block_n)
    return out


# speedup vs baseline: 7.4590x; 1.6691x over previous
import functools

import jax
import jax.numpy as jnp
from jax import lax
from jax.experimental import pallas as pl
from jax.experimental.pallas import tpu as pltpu
from jax.experimental.pallas import tpu_sc as plsc

NUM_ITERATIONS = 2
ALPHA = 0.5
LAMBDA_GRAPH = 0.1
LN_EPS = 1e-5

NC = 2
NS = 16
NW = NC * NS


def _sc_degree(col3, ones_ch, zeros_nd):
    NCHUNK, CH = col3.shape[1], col3.shape[2]
    N, D = zeros_nd.shape
    RB = 1000
    NRB = N // RB

    @functools.partial(
        pl.kernel,
        out_type=jax.ShapeDtypeStruct((NC, N, D), jnp.float32),
        mesh=plsc.VectorSubcoreMesh(core_axis_name="c", subcore_axis_name="s"),
        scratch_types=[
            pltpu.VMEM((NCHUNK, CH), jnp.int32),
            pltpu.VMEM((CH, D), jnp.float32),
            pltpu.VMEM_SHARED((N, D), jnp.float32),
            pltpu.SemaphoreType.DMA,
        ],
    )
    def k(col_hbm, ones_hbm, z_hbm, out_hbm, idxc, onesv, acc, ssem):
        c = lax.axis_index("c")
        s = lax.axis_index("s")
        wid = c * NS + s
        pltpu.sync_copy(col_hbm.at[wid], idxc)
        pltpu.sync_copy(ones_hbm, onesv)

        @pl.when(s < NRB)
        def _():
            pltpu.sync_copy(z_hbm.at[pl.ds(s * RB, RB)],
                            acc.at[pl.ds(s * RB, RB)])

        plsc.subcore_barrier()

        GD = 25

        def group(g, carry):
            def fire(t, c2):
                pltpu.async_copy(onesv, acc.at[idxc.at[g * GD + t]], ssem,
                                 add=True)
                return c2

            lax.fori_loop(0, GD, fire, 0)

            def drain(t, c2):
                pltpu.make_async_copy(onesv, acc.at[idxc.at[g * GD + t]],
                                      ssem).wait()
                return c2

            lax.fori_loop(0, GD, drain, 0)
            return carry

        lax.fori_loop(0, NCHUNK // GD, group, 0)
        plsc.subcore_barrier()

        @pl.when(s < NRB)
        def _():
            pltpu.sync_copy(acc.at[pl.ds(s * RB, RB)],
                            out_hbm.at[c, pl.ds(s * RB, RB)])

    return k(col3, ones_ch, zeros_nd)


def _sc_scatter(row4, col4, y, zeros_nd):
    IP, IC, CH = row4.shape[1], row4.shape[2], row4.shape[3]
    N, D = y.shape
    RB = 1000
    NRB = N // RB

    @functools.partial(
        pl.kernel,
        out_type=jax.ShapeDtypeStruct((NC, N, D), jnp.float32),
        mesh=plsc.VectorSubcoreMesh(core_axis_name="c", subcore_axis_name="s"),
        scratch_types=[
            pltpu.VMEM((IC, CH), jnp.int32),
            pltpu.VMEM((IC, CH), jnp.int32),
            pltpu.VMEM((CH, D), jnp.float32),
            pltpu.VMEM((CH, D), jnp.float32),
            pltpu.VMEM_SHARED((N, D), jnp.float32),
            pltpu.SemaphoreType.DMA,
            pltpu.SemaphoreType.DMA,
        ],
    )
    def k(row_hbm, col_hbm, y_hbm, z_hbm, out_hbm, idxr, idxc, buf0, buf1,
          acc, gsem0, gsem1):
        c = lax.axis_index("c")
        s = lax.axis_index("s")
        wid = c * NS + s

        @pl.when(s < NRB)
        def _():
            pltpu.sync_copy(z_hbm.at[pl.ds(s * RB, RB)],
                            acc.at[pl.ds(s * RB, RB)])

        plsc.subcore_barrier()

        def window(p, carry):
            pltpu.sync_copy(row_hbm.at[wid, p], idxr)
            pltpu.sync_copy(col_hbm.at[wid, p], idxc)
            pltpu.async_copy(y_hbm.at[idxc.at[0]], buf0, gsem0)
            pltpu.async_copy(y_hbm.at[idxc.at[1]], buf1, gsem1)

            def body(i, c2):
                j0 = 2 * i
                j1 = j0 + 1
                pltpu.make_async_copy(y_hbm.at[idxc.at[j0]], buf0,
                                      gsem0).wait()
                pltpu.sync_copy(buf0, acc.at[idxr.at[j0]], add=True)

                @pl.when(j0 + 2 < IC)
                def _():
                    pltpu.async_copy(y_hbm.at[idxc.at[j0 + 2]], buf0, gsem0)

                pltpu.make_async_copy(y_hbm.at[idxc.at[j1]], buf1,
                                      gsem1).wait()
                pltpu.sync_copy(buf1, acc.at[idxr.at[j1]], add=True)

                @pl.when(j1 + 2 < IC)
                def _():
                    pltpu.async_copy(y_hbm.at[idxc.at[j1 + 2]], buf1, gsem1)

                return c2

            lax.fori_loop(0, IC // 2, body, 0)
            if IC % 2:
                jt = IC - 1
                pltpu.make_async_copy(y_hbm.at[idxc.at[jt]], buf0,
                                      gsem0).wait()
                pltpu.sync_copy(buf0, acc.at[idxr.at[jt]], add=True)
            return carry

        lax.fori_loop(0, IP, window, 0)
        plsc.subcore_barrier()

        @pl.when(s < NRB)
        def _():
            pltpu.sync_copy(acc.at[pl.ds(s * RB, RB)],
                            out_hbm.at[c, pl.ds(s * RB, RB)])

    return k(row4, col4, y, zeros_nd)


def _retrieve_block(xb, wq, keys, values, beta):
    q = jnp.dot(xb, wq, preferred_element_type=jnp.float32)
    logits = beta * lax.dot_general(
        q, keys, (((1,), (1,)), ((), ())), preferred_element_type=jnp.float32)
    m = jnp.max(logits, axis=-1, keepdims=True)
    e = jnp.exp(logits - m)
    attn = e / jnp.sum(e, axis=-1, keepdims=True)
    return jnp.dot(attn, values, preferred_element_type=jnp.float32)


def _tc_prep(x, degp, W_q, keys, values, beta2, block_n):
    N, D = x.shape
    K = keys.shape[0]

    def body(x_ref, deg_ref, wq_ref, k_ref, v_ref, b_ref,
             dis_ref, y_ref, r_ref):
        xb = x_ref[...]
        deg = deg_ref[0, :, 0:1] + deg_ref[1, :, 0:1]
        dis = jnp.where(deg > 0, lax.rsqrt(deg), 0.0)
        dis_ref[...] = dis
        y_ref[...] = xb * dis
        r_ref[...] = _retrieve_block(xb, wq_ref[...], k_ref[...], v_ref[...],
                                     b_ref[0, 0])

    return pl.pallas_call(
        body,
        grid=(N // block_n,),
        in_specs=[
            pl.BlockSpec((block_n, D), lambda i: (i, 0)),
            pl.BlockSpec((NC, block_n, D), lambda i: (0, i, 0)),
            pl.BlockSpec((D, D), lambda i: (0, 0)),
            pl.BlockSpec((K, D), lambda i: (0, 0)),
            pl.BlockSpec((K, D), lambda i: (0, 0)),
            pl.BlockSpec(memory_space=pltpu.SMEM),
        ],
        out_specs=[
            pl.BlockSpec((block_n, 1), lambda i: (i, 0)),
            pl.BlockSpec((block_n, D), lambda i: (i, 0)),
            pl.BlockSpec((block_n, D), lambda i: (i, 0)),
        ],
        out_shape=[
            jax.ShapeDtypeStruct((N, 1), jnp.float32),
            jax.ShapeDtypeStruct((N, D), jnp.float32),
            jax.ShapeDtypeStruct((N, D), jnp.float32),
        ],
    )(x, degp, W_q, keys, values, beta2)


def _update_block(xb, rb, aggp, dis):
    agg = dis * (aggp[0] + aggp[1])
    lap = xb - agg
    return (1.0 - ALPHA) * xb + ALPHA * (rb - 2.0 * LAMBDA_GRAPH * lap)


def _tc_update_retrieve(x, r, aggp, dis, W_q, keys, values, beta2, block_n):
    N, D = x.shape
    K = keys.shape[0]

    def body(x_ref, r_ref, agg_ref, dis_ref, wq_ref, k_ref, v_ref, b_ref,
             xn_ref, y_ref, rn_ref):
        dis = dis_ref[...]
        xn = _update_block(x_ref[...], r_ref[...], agg_ref[...], dis)
        xn_ref[...] = xn
        y_ref[...] = xn * dis
        rn_ref[...] = _retrieve_block(xn, wq_ref[...], k_ref[...], v_ref[...],
                                      b_ref[0, 0])

    return pl.pallas_call(
        body,
        grid=(N // block_n,),
        in_specs=[
            pl.BlockSpec((block_n, D), lambda i: (i, 0)),
            pl.BlockSpec((block_n, D), lambda i: (i, 0)),
            pl.BlockSpec((NC, block_n, D), lambda i: (0, i, 0)),
            pl.BlockSpec((block_n, 1), lambda i: (i, 0)),
            pl.BlockSpec((D, D), lambda i: (0, 0)),
            pl.BlockSpec((K, D), lambda i: (0, 0)),
            pl.BlockSpec((K, D), lambda i: (0, 0)),
            pl.BlockSpec(memory_space=pltpu.SMEM),
        ],
        out_specs=[
            pl.BlockSpec((block_n, D), lambda i: (i, 0)),
            pl.BlockSpec((block_n, D), lambda i: (i, 0)),
            pl.BlockSpec((block_n, D), lambda i: (i, 0)),
        ],
        out_shape=[
            jax.ShapeDtypeStruct((N, D), jnp.float32),
            jax.ShapeDtypeStruct((N, D), jnp.float32),
            jax.ShapeDtypeStruct((N, D), jnp.float32),
        ],
    )(x, r, aggp, dis, W_q, keys, values, beta2)


def _tc_final(x, r, aggp, dis, ln_gamma, ln_beta, block_n):
    N, D = x.shape

    def body(x_ref, r_ref, agg_ref, dis_ref, g_ref, b_ref, out_ref):
        xn = _update_block(x_ref[...], r_ref[...], agg_ref[...], dis_ref[...])
        mean = jnp.mean(xn, axis=-1, keepdims=True)
        d = xn - mean
        var = jnp.mean(d * d, axis=-1, keepdims=True)
        out_ref[...] = g_ref[...] * d * lax.rsqrt(var + LN_EPS) + b_ref[...]

    return pl.pallas_call(
        body,
        grid=(N // block_n,),
        in_specs=[
            pl.BlockSpec((block_n, D), lambda i: (i, 0)),
            pl.BlockSpec((block_n, D), lambda i: (i, 0)),
            pl.BlockSpec((NC, block_n, D), lambda i: (0, i, 0)),
            pl.BlockSpec((block_n, 1), lambda i: (i, 0)),
            pl.BlockSpec((D,), lambda i: (0,)),
            pl.BlockSpec((D,), lambda i: (0,)),
        ],
        out_specs=pl.BlockSpec((block_n, D), lambda i: (i, 0)),
        out_shape=jax.ShapeDtypeStruct((N, D), jnp.float32),
    )(x, r, aggp, dis, ln_gamma, ln_beta)


def kernel(x, edge_index, W_q, keys_mem, values_mem, beta, ln_gamma, ln_beta):
    N, D = x.shape
    E = edge_index.shape[1]
    EPW = E // NW
    CH = 80
    NCHUNK = EPW // CH
    IP = 5
    IC = NCHUNK // IP

    ei = edge_index.astype(jnp.int32)
    row4 = ei[0].reshape(NW, IP, IC, CH)
    col4 = ei[1].reshape(NW, IP, IC, CH)
    col3 = ei[1].reshape(NW, NCHUNK, CH)
    zeros_nd = jnp.zeros((N, D), jnp.float32)
    ones_ch = jnp.ones((CH, D), jnp.float32)
    beta2 = jnp.reshape(beta.astype(jnp.float32), (1, 1))
    block_n = 2000

    degp = _sc_degree(col3, ones_ch, zeros_nd)
    dis, y, r = _tc_prep(x, degp, W_q, keys_mem, values_mem, beta2, block_n)
    for it in range(NUM_ITERATIONS):
        aggp = _sc_scatter(row4, col4, y, zeros_nd)
        if it + 1 < NUM_ITERATIONS:
            x, y, r = _tc_update_retrieve(
                x, r, aggp, dis, W_q, keys_mem, values_mem, beta2, block_n)
        else:
            out = _tc_final(x, r, aggp, dis, ln_gamma, ln_beta, block_n)
    return out
